# baseline (speedup 1.0000x reference)
import functools

import jax
import jax.numpy as jnp
from jax import lax
from jax.experimental import pallas as pl
from jax.experimental.pallas import tpu as pltpu
from jax.experimental.pallas import tpu_sc as plsc

_VOCAB = 1000000
_DIM = 64
_B = 16384
_CTX = 20
_LANES = 16

_NW = 32
_TASKS = 2 * _B
_TPW = _TASKS // _NW
_CB = 4
_ROWS = _CB * _CTX
_NCHUNK = _TPW // _CB
_OFFW = 96
_UBLK = 32
_NUB = _TPW // _UBLK
_CPU = _UBLK // (2 * _CB)
_MROWS = _TPW // 8


def _sc_body(vdiv_hbm, voffc_hbm, udiv_hbm, vtab_hbm, utab_hbm,
             p_hbm, vdiv_v, voffc_v, udiv_v, ubuf0, ubuf1,
             vbuf0, vbuf1, pbuf, vsem0, vsem1, usem0, usem1, psem):
    cid = lax.axis_index("c")
    sid = lax.axis_index("s")
    wid = sid * 2 + cid

    pltpu.sync_copy(vdiv_hbm.at[wid], vdiv_v)
    pltpu.sync_copy(voffc_hbm.at[wid], voffc_v)
    pltpu.sync_copy(udiv_hbm.at[wid], udiv_v)

    pltpu.async_copy(utab_hbm.at[udiv_v.at[0]], ubuf0, usem0)
    pltpu.async_copy(vtab_hbm.at[vdiv_v.at[0]], vbuf0, vsem0)

    sign = jnp.where(wid < _NW // 2, 1.0, -1.0).astype(jnp.float32)
    scale = sign * (1.0 / _CTX)

    vbufs = (vbuf0, vbuf1)
    vsems = (vsem0, vsem1)
    ubufs = (ubuf0, ubuf1)
    usems = (usem0, usem1)

    @pl.loop(0, _NUB // 2)
    def _outer(uu):
        for u2 in range(2):
            ublk = uu * 2 + u2

            pltpu.make_async_copy(utab_hbm.at[udiv_v.at[ublk]],
                                  ubufs[u2], usems[u2]).wait()

            @pl.when(ublk + 1 < _NUB)
            def _():
                nb = jnp.minimum(ublk + 1, _NUB - 1)
                pltpu.async_copy(utab_hbm.at[udiv_v.at[nb]],
                                 ubufs[1 - u2], usems[1 - u2])

            ubuf = ubufs[u2]

            @pl.loop(0, _CPU)
            def _inner(cc):
                mg = ublk * _CPU + cc
                for b in range(2):
                    chunk = mg * 2 + b
                    nxt = chunk + 1

                    @pl.when(nxt < _NCHUNK)
                    def _():
                        nrow = jnp.minimum(nxt, _NCHUNK - 1)
                        pltpu.async_copy(vtab_hbm.at[vdiv_v.at[nrow]],
                                         vbufs[1 - b], vsems[1 - b])

                    pltpu.make_async_copy(vtab_hbm.at[vdiv_v.at[chunk]],
                                          vbufs[b], vsems[b]).wait()

                    buf = vbufs[b]
                    offv = [voffc_v[chunk, pl.ds(k * _LANES, _LANES)]
                            for k in range(_ROWS // _LANES)]
                    uov = voffc_v[chunk, pl.ds(_ROWS, _LANES)]
                    for t in range(_CB):
                        lrow = (cc * 2 + b) * _CB + t
                        uo = uov[t]
                        p = None
                        for d in range(_DIM // _LANES):
                            acc = None
                            for c in range(_CTX):
                                r = t * _CTX + c
                                off = offv[r // _LANES][r % _LANES] + d * _LANES
                                x = buf[r, pl.ds(off, _LANES)]
                                acc = x if acc is None else acc + x
                            urow = ubuf[lrow, pl.ds(uo + d * _LANES, _LANES)]
                            term = acc * urow
                            p = term if p is None else p + term
                        pbuf[mg, pl.ds((_CB * b + t) * _LANES, _LANES)] = \
                            p * scale
                pltpu.async_copy(pbuf.at[mg], p_hbm.at[wid, mg], psem)

    @pl.loop(0, _MROWS)
    def _drain(i):
        pltpu.make_async_copy(pbuf.at[0], p_hbm.at[wid, 0], psem).wait()


@functools.cache
def _sc_pdots():
    return pl.kernel(
        _sc_body,
        out_type=jax.ShapeDtypeStruct((_NW, _MROWS, 8 * _LANES),
                                      jnp.float32),
        mesh=plsc.VectorSubcoreMesh(core_axis_name="c", subcore_axis_name="s",
                                    num_cores=2, num_subcores=16),
        compiler_params=pltpu.CompilerParams(use_tc_tiling_on_sc=True),
        scratch_types=[
            pltpu.VMEM((_NCHUNK, _ROWS), jnp.int32),
            pltpu.VMEM((_NCHUNK, _OFFW), jnp.int32),
            pltpu.VMEM((_NUB, _UBLK), jnp.int32),
            pltpu.VMEM((_UBLK, 2 * _DIM), jnp.float32),
            pltpu.VMEM((_UBLK, 2 * _DIM), jnp.float32),
            pltpu.VMEM((_ROWS, 2 * _DIM), jnp.float32),
            pltpu.VMEM((_ROWS, 2 * _DIM), jnp.float32),
            pltpu.VMEM((_MROWS, 8 * _LANES), jnp.float32),
            pltpu.SemaphoreType.DMA,
            pltpu.SemaphoreType.DMA,
            pltpu.SemaphoreType.DMA,
            pltpu.SemaphoreType.DMA,
            pltpu.SemaphoreType.DMA,
        ],
    )


def _loss_body(p_ref, out_ref):
    x = p_ref[...]
    j = lax.broadcasted_iota(jnp.int32, (8 * _LANES, 8), 0)
    t = lax.broadcasted_iota(jnp.int32, (8 * _LANES, 8), 1)
    sel = (j // _LANES == t).astype(jnp.float32)
    z = jnp.dot(x, sel, preferred_element_type=jnp.float32)
    out_ref[0, 0] = -jnp.sum(jax.nn.log_sigmoid(z))


_loss_call = pl.pallas_call(
    _loss_body,
    out_shape=jax.ShapeDtypeStruct((1, 1), jnp.float32),
    out_specs=pl.BlockSpec(memory_space=pltpu.SMEM),
)


def kernel(pos_v, pos_u, neg_v, neg_u, v_table, u_table):
    vidx = jnp.concatenate([pos_v.astype(jnp.int32).reshape(-1),
                            neg_v.astype(jnp.int32).reshape(-1)])
    uidx = jnp.concatenate([pos_u.astype(jnp.int32),
                            neg_u.astype(jnp.int32)])
    vdiv = (vidx >> 1).reshape(_NW, _NCHUNK, _ROWS)
    voff = ((vidx & 1) * _DIM).reshape(_NW, _NCHUNK, _ROWS)
    uoff = ((uidx & 1) * _DIM).reshape(_NW, _NCHUNK, _CB)
    pad = jnp.zeros((_NW, _NCHUNK, _OFFW - _ROWS - _CB), jnp.int32)
    voffc = jnp.concatenate([voff, uoff, pad], axis=2)
    udiv = (uidx >> 1).reshape(_NW, _NUB, _UBLK)
    vtab = v_table.reshape(_VOCAB // 2, 2 * _DIM)
    utab = u_table.reshape(_VOCAB // 2, 2 * _DIM)
    p = _sc_pdots()(vdiv, voffc, udiv, vtab, utab)
    loss = _loss_call(p.reshape(_TASKS // 8, 8 * _LANES))
    return loss[0, 0]


# speedup vs baseline: 1.0465x; 1.0465x over previous
import functools

import jax
import jax.numpy as jnp
from jax import lax
from jax.experimental import pallas as pl
from jax.experimental.pallas import tpu as pltpu
from jax.experimental.pallas import tpu_sc as plsc

_VOCAB = 1000000
_DIM = 64
_B = 16384
_CTX = 20
_LANES = 16

_NW = 32
_TASKS = 2 * _B
_TPW = _TASKS // _NW
_CB = 4
_ROWS = _CB * _CTX
_NCHUNK = _TPW // _CB
_UCH = 128
_NUCH = _TPW // _UCH


def _sc_body(vidx_hbm, uidx_hbm, vtab_hbm, utab_hbm, p_hbm,
             vidx_v, uidx_v, urows_v, vbuf0, vbuf1, pbuf,
             sem0, sem1, semu):
    cid = lax.axis_index("c")
    sid = lax.axis_index("s")
    wid = sid * 2 + cid

    pltpu.sync_copy(vidx_hbm.at[wid], vidx_v)
    pltpu.sync_copy(uidx_hbm.at[wid], uidx_v)

    for k in range(_NUCH):
        pltpu.async_copy(utab_hbm.at[uidx_v.at[k]],
                         urows_v.at[pl.ds(k * _UCH, _UCH)], semu)
    pltpu.async_copy(vtab_hbm.at[vidx_v.at[0]], vbuf0, sem0)
    for k in range(_NUCH):
        pltpu.make_async_copy(utab_hbm.at[uidx_v.at[k]],
                              urows_v.at[pl.ds(k * _UCH, _UCH)], semu).wait()

    sign = jnp.where(wid < _NW // 2, 1.0, -1.0).astype(jnp.float32)
    scale = sign * (1.0 / _CTX)

    vbufs = (vbuf0, vbuf1)
    sems = (sem0, sem1)

    @pl.loop(0, _NCHUNK // 2)
    def _chunk_loop(jj):
        for b in range(2):
            chunk = jj * 2 + b
            nxt = chunk + 1

            @pl.when(nxt < _NCHUNK)
            def _():
                nrow = jnp.minimum(nxt, _NCHUNK - 1)
                pltpu.async_copy(vtab_hbm.at[vidx_v.at[nrow]],
                                 vbufs[1 - b], sems[1 - b])

            pltpu.make_async_copy(vtab_hbm.at[vidx_v.at[chunk]],
                                  vbufs[b], sems[b]).wait()

            buf = vbufs[b]
            for t in range(_CB):
                task = chunk * _CB + t
                p = None
                for d in range(_DIM // _LANES):
                    sl = pl.ds(d * _LANES, _LANES)
                    acc = buf[t * _CTX, sl]
                    for c in range(1, _CTX):
                        acc = acc + buf[t * _CTX + c, sl]
                    term = acc * urows_v[task, sl]
                    p = term if p is None else p + term
                pbuf[task] = p * scale

    pltpu.sync_copy(pbuf, p_hbm.at[wid])


@functools.cache
def _sc_zdots():
    return pl.kernel(
        _sc_body,
        out_type=jax.ShapeDtypeStruct((_NW, _TPW, _LANES), jnp.float32),
        mesh=plsc.VectorSubcoreMesh(core_axis_name="c", subcore_axis_name="s",
                                    num_cores=2, num_subcores=16),
        compiler_params=pltpu.CompilerParams(use_tc_tiling_on_sc=False),
        scratch_types=[
            pltpu.VMEM((_NCHUNK, _ROWS), jnp.int32),
            pltpu.VMEM((_NUCH, _UCH), jnp.int32),
            pltpu.VMEM((_TPW, _DIM), jnp.float32),
            pltpu.VMEM((_ROWS, _DIM), jnp.float32),
            pltpu.VMEM((_ROWS, _DIM), jnp.float32),
            pltpu.VMEM((_TPW, _LANES), jnp.float32),
            pltpu.SemaphoreType.DMA,
            pltpu.SemaphoreType.DMA,
            pltpu.SemaphoreType.DMA,
        ],
    )


def _loss_body(p_ref, out_ref):
    x = p_ref[...]
    j = lax.broadcasted_iota(jnp.int32, (16 * _LANES, _LANES), 0)
    t = lax.broadcasted_iota(jnp.int32, (16 * _LANES, _LANES), 1)
    sel = (j // _LANES == t).astype(jnp.float32)
    z = jnp.dot(x, sel, preferred_element_type=jnp.float32)
    out_ref[0, 0] = -jnp.sum(jax.nn.log_sigmoid(z))


_loss_call = pl.pallas_call(
    _loss_body,
    out_shape=jax.ShapeDtypeStruct((1, 1), jnp.float32),
    out_specs=pl.BlockSpec(memory_space=pltpu.SMEM),
)


def kernel(pos_v, pos_u, neg_v, neg_u, v_table, u_table):
    vidx = jnp.concatenate([pos_v.astype(jnp.int32).reshape(-1),
                            neg_v.astype(jnp.int32).reshape(-1)])
    vidx = vidx.reshape(_NW, _NCHUNK, _ROWS)
    uidx = jnp.concatenate([pos_u.astype(jnp.int32),
                            neg_u.astype(jnp.int32)])
    uidx = uidx.reshape(_NW, _NUCH, _UCH)
    p = _sc_zdots()(vidx, uidx, v_table, u_table)
    loss = _loss_call(p.reshape(_TASKS // _LANES, _LANES * _LANES))
    return loss[0, 0]


# speedup vs baseline: 1.1792x; 1.1268x over previous
import functools

import jax
import jax.numpy as jnp
from jax import lax
from jax.experimental import pallas as pl
from jax.experimental.pallas import tpu as pltpu
from jax.experimental.pallas import tpu_sc as plsc

_VOCAB = 1000000
_DIM = 64
_B = 16384
_CTX = 20
_LANES = 16

_NW = 32
_TASKS = 2 * _B
_TPW = _TASKS // _NW
_CB = 4
_ROWS = _CB * _CTX
_NCHUNK = _TPW // _CB
_UBLK = 32
_NUB = _TPW // _UBLK
_CPU = _UBLK // (2 * _CB)
_MROWS = _TPW // 8


def _sc_body(vdiv_hbm, udiv_hbm, wtab_hbm, p_hbm,
             vdiv_v, udiv_v, ubuf0, ubuf1,
             vbuf0, vbuf1, pbuf, vsem0, vsem1, usem0, usem1, psem):
    cid = lax.axis_index("c")
    sid = lax.axis_index("s")
    wid = sid * 2 + cid

    pltpu.sync_copy(vdiv_hbm.at[wid], vdiv_v)
    pltpu.sync_copy(udiv_hbm.at[wid], udiv_v)

    pltpu.async_copy(wtab_hbm.at[udiv_v.at[0]], ubuf0, usem0)
    pltpu.async_copy(wtab_hbm.at[vdiv_v.at[0]], vbuf0, vsem0)

    sign = jnp.where(wid < _NW // 2, 1.0, -1.0).astype(jnp.float32)
    scale = sign * (1.0 / _CTX)

    vbufs = (vbuf0, vbuf1)
    vsems = (vsem0, vsem1)
    ubufs = (ubuf0, ubuf1)
    usems = (usem0, usem1)

    @pl.loop(0, _NUB // 2)
    def _outer(uu):
        for u2 in range(2):
            ublk = uu * 2 + u2

            pltpu.make_async_copy(wtab_hbm.at[udiv_v.at[ublk]],
                                  ubufs[u2], usems[u2]).wait()

            @pl.when(ublk + 1 < _NUB)
            def _():
                nb = jnp.minimum(ublk + 1, _NUB - 1)
                pltpu.async_copy(wtab_hbm.at[udiv_v.at[nb]],
                                 ubufs[1 - u2], usems[1 - u2])

            ubuf = ubufs[u2]

            @pl.loop(0, _CPU)
            def _inner(cc):
                mg = ublk * _CPU + cc
                for b in range(2):
                    chunk = mg * 2 + b
                    nxt = chunk + 1

                    @pl.when(nxt < _NCHUNK)
                    def _():
                        nrow = jnp.minimum(nxt, _NCHUNK - 1)
                        pltpu.async_copy(wtab_hbm.at[vdiv_v.at[nrow]],
                                         vbufs[1 - b], vsems[1 - b])

                    pltpu.make_async_copy(wtab_hbm.at[vdiv_v.at[chunk]],
                                          vbufs[b], vsems[b]).wait()

                    buf = vbufs[b]
                    for t in range(_CB):
                        lrow = (cc * 2 + b) * _CB + t
                        p = None
                        for d in range(_DIM // _LANES):
                            sl = pl.ds(d * _LANES, _LANES)
                            acc = None
                            for c in range(_CTX):
                                x = buf[t * _CTX + c, sl]
                                acc = x if acc is None else acc + x
                            urow = ubuf[lrow,
                                        pl.ds(_DIM + d * _LANES, _LANES)]
                            term = acc * urow
                            p = term if p is None else p + term
                        pbuf[mg, pl.ds((_CB * b + t) * _LANES, _LANES)] = \
                            p * scale
                pltpu.async_copy(pbuf.at[mg], p_hbm.at[wid, mg], psem)

    @pl.loop(0, _MROWS)
    def _drain(i):
        pltpu.make_async_copy(pbuf.at[0], p_hbm.at[wid, 0], psem).wait()


@functools.cache
def _sc_pdots():
    return pl.kernel(
        _sc_body,
        out_type=jax.ShapeDtypeStruct((_NW, _MROWS, 8 * _LANES),
                                      jnp.float32),
        mesh=plsc.VectorSubcoreMesh(core_axis_name="c", subcore_axis_name="s",
                                    num_cores=2, num_subcores=16),
        compiler_params=pltpu.CompilerParams(use_tc_tiling_on_sc=True),
        scratch_types=[
            pltpu.VMEM((_NCHUNK, _ROWS), jnp.int32),
            pltpu.VMEM((_NUB, _UBLK), jnp.int32),
            pltpu.VMEM((_UBLK, 2 * _DIM), jnp.float32),
            pltpu.VMEM((_UBLK, 2 * _DIM), jnp.float32),
            pltpu.VMEM((_ROWS, 2 * _DIM), jnp.float32),
            pltpu.VMEM((_ROWS, 2 * _DIM), jnp.float32),
            pltpu.VMEM((_MROWS, 8 * _LANES), jnp.float32),
            pltpu.SemaphoreType.DMA,
            pltpu.SemaphoreType.DMA,
            pltpu.SemaphoreType.DMA,
            pltpu.SemaphoreType.DMA,
            pltpu.SemaphoreType.DMA,
        ],
    )


def _loss_body(p_ref, out_ref):
    x = p_ref[...]
    j = lax.broadcasted_iota(jnp.int32, (8 * _LANES, 8), 0)
    t = lax.broadcasted_iota(jnp.int32, (8 * _LANES, 8), 1)
    sel = (j // _LANES == t).astype(jnp.float32)
    z = jnp.dot(x, sel, preferred_element_type=jnp.float32)
    out_ref[0, 0] = -jnp.sum(jax.nn.log_sigmoid(z))


_loss_call = pl.pallas_call(
    _loss_body,
    out_shape=jax.ShapeDtypeStruct((1, 1), jnp.float32),
    out_specs=pl.BlockSpec(memory_space=pltpu.SMEM),
)


def kernel(pos_v, pos_u, neg_v, neg_u, v_table, u_table):
    vidx = jnp.concatenate([pos_v.astype(jnp.int32).reshape(-1),
                            neg_v.astype(jnp.int32).reshape(-1)])
    uidx = jnp.concatenate([pos_u.astype(jnp.int32),
                            neg_u.astype(jnp.int32)])
    vdiv = vidx.reshape(_NW, _NCHUNK, _ROWS)
    udiv = uidx.reshape(_NW, _NUB, _UBLK)
    wtab = jnp.concatenate([v_table, u_table], axis=1)
    p = _sc_pdots()(vdiv, udiv, wtab)
    loss = _loss_call(p.reshape(_TASKS // 8, 8 * _LANES))
    return loss[0, 0]
